# baseline (speedup 1.0000x reference)
import jax
import jax.numpy as jnp
from jax import lax
from jax.experimental import pallas as pl
from jax.experimental.pallas import tpu as pltpu
from jax.experimental.pallas import tpu_sc as plsc
import functools

N = 320000
D = 128
NSEG = 10000

NC = 2
NS = 16
CHUNK = 160
NCH = N // CHUNK
NBUF = 4
SEG_SPLIT = 5056
HALF0 = SEG_SPLIT
HALF1 = NSEG - SEG_SPLIT
ACC_ROWS = 5064
ZROWS = 56


def _sc_segment_sum(rows, ids, split):
    mesh = plsc.VectorSubcoreMesh(core_axis_name="c", subcore_axis_name="s")

    @functools.partial(
        pl.kernel,
        out_type=jax.ShapeDtypeStruct((NSEG, D), jnp.float32),
        mesh=mesh,
        scratch_types=(
            [pltpu.VMEM((CHUNK, D), jnp.float32)] * NBUF
            + [pltpu.VMEM((CHUNK,), jnp.int32)] * NBUF
            + [pltpu.VMEM((ZROWS, D), jnp.float32),
               pltpu.VMEM((16,), jnp.int32),
               pltpu.VMEM_SHARED((ACC_ROWS, D), jnp.float32)]
            + [pltpu.SemaphoreType.DMA] * (3 * NBUF)
        ),
    )
    def body(rows_hbm, ids_hbm, split_hbm, out_hbm, *refs):
        rows_v = refs[0:NBUF]
        idx_v = refs[NBUF:2 * NBUF]
        zbuf, split_v, acc = refs[2 * NBUF:2 * NBUF + 3]
        sems = refs[2 * NBUF + 3:]
        rsem = sems[0:NBUF]
        isem = sems[NBUF:2 * NBUF]
        ssem = sems[2 * NBUF:3 * NBUF]
        cid = lax.axis_index("c")
        sid = lax.axis_index("s")

        pltpu.sync_copy(split_hbm, split_v)
        r1 = split_v[...][0]
        ca = (r1 + CHUNK - 1) // CHUNK
        cb = r1 // CHUNK
        first = jnp.where(cid == 0, 0, cb)
        limit = jnp.where(cid == 0, ca, NCH - cb)
        nsteps = jnp.maximum(0, (limit - sid + NS - 1) // NS)

        half_lo = jnp.where(cid == 0, 0, SEG_SPLIT)
        half_len = jnp.where(cid == 0, HALF0, HALF1)
        lo_v = jnp.full((16,), half_lo, jnp.int32)
        len_v = jnp.full((16,), half_len, jnp.int32)

        def chunk_of(k):
            return first + sid + NS * k

        def start(k, b):
            off = pl.multiple_of(chunk_of(k) * CHUNK, CHUNK)
            pltpu.async_copy(ids_hbm.at[pl.ds(off, CHUNK)], idx_v[b], isem[b])
            pltpu.async_copy(rows_hbm.at[pl.ds(off, CHUNK)], rows_v[b], rsem[b])

        def wait(b):
            pltpu.make_async_copy(ids_hbm.at[pl.ds(0, CHUNK)], idx_v[b], isem[b]).wait()
            pltpu.make_async_copy(rows_hbm.at[pl.ds(0, CHUNK)], rows_v[b], rsem[b]).wait()

        def localize(b):
            for j in range(CHUNK // 16):
                v = idx_v[b][pl.ds(j * 16, 16)]
                loc = v - lo_v
                ok = (loc >= 0) & (loc < len_v)
                idx_v[b][pl.ds(j * 16, 16)] = jnp.where(ok, loc, len_v)

        def scatter(b):
            pltpu.async_copy(rows_v[b], acc.at[idx_v[b]], ssem[b], add=True)

        def wait_scatter(b):
            pltpu.make_async_copy(rows_v[b], acc.at[idx_v[b]], ssem[b]).wait()

        @pl.when(nsteps > 0)
        def _():
            start(0, 0)

        zvec = jnp.zeros((16,), jnp.float32)

        def zstore(i, carry):
            r = i // 8
            c = lax.rem(i, 8) * 16
            zbuf[r, pl.ds(c, 16)] = zvec
            return carry

        lax.fori_loop(0, ZROWS * 8, zstore, 0)

        zstart = sid * 320
        zlen = jnp.clip(ACC_ROWS - zstart, 0, 320)
        nz = zlen // ZROWS

        def zcopy(i, carry):
            dst = pl.multiple_of(zstart + i * ZROWS, 8)
            pltpu.sync_copy(zbuf, acc.at[pl.ds(dst, ZROWS)])
            return carry

        lax.fori_loop(0, nz, zcopy, 0)
        zdst = pl.multiple_of(zstart + nz * ZROWS, 8)
        pltpu.sync_copy(zbuf.at[pl.ds(0, 40)], acc.at[pl.ds(zdst, 40)])

        plsc.subcore_barrier()

        def step(k, b):
            wait(b)
            localize(b)
            scatter(b)
            nb = (b + 1) % NBUF

            @pl.when(k + 1 < nsteps)
            def _():
                @pl.when(k >= NBUF - 1)
                def _():
                    wait_scatter(nb)

                start(k + 1, nb)

        def quad(q, carry):
            for b in range(NBUF):
                step(NBUF * q + b, b)
            return carry

        lax.fori_loop(0, nsteps // NBUF, quad, 0)
        tbase = (nsteps // NBUF) * NBUF
        for r in range(NBUF - 1):
            @pl.when(tbase + r < nsteps)
            def _(r=r):
                step(tbase + r, r)

        for b in range(NBUF):
            @pl.when(nsteps > b)
            def _(b=b):
                wait_scatter(b)

        plsc.subcore_barrier()

        obase = pl.multiple_of(sid * 320, 8)

        @pl.when(sid < NS - 1)
        def _():
            pltpu.sync_copy(acc.at[pl.ds(obase, 320)],
                            out_hbm.at[pl.ds(pl.multiple_of(half_lo + obase, 8), 320)])

        @pl.when((sid == NS - 1) & (cid == 0))
        def _():
            pltpu.sync_copy(acc.at[pl.ds(4800, 256)],
                            out_hbm.at[pl.ds(4800, 256)])

        @pl.when((sid == NS - 1) & (cid == 1))
        def _():
            pltpu.sync_copy(acc.at[pl.ds(4800, 144)],
                            out_hbm.at[pl.ds(SEG_SPLIT + 4800, 144)])

    return body(rows, ids, split)


def kernel(output, batch):
    ids = batch.astype(jnp.int32)
    r1 = jnp.sum((ids < SEG_SPLIT).astype(jnp.int32)).astype(jnp.int32)
    split = jnp.broadcast_to(r1, (16,))
    return _sc_segment_sum(output, ids, split).reshape(-1)


# speedup vs baseline: 1.0844x; 1.0844x over previous
import jax
import jax.numpy as jnp
from jax import lax
from jax.experimental import pallas as pl
from jax.experimental.pallas import tpu as pltpu
from jax.experimental.pallas import tpu_sc as plsc
import functools

N = 320000
D = 128
NSEG = 10000

NC = 2
NS = 16
CHUNK = 320
NCH = N // CHUNK
NBUF = 2
SEG_SPLIT = 5056
HALF0 = SEG_SPLIT
HALF1 = NSEG - SEG_SPLIT
ACC_ROWS = 5064
ZROWS = 56


def _sc_segment_sum(rows, ids, split):
    mesh = plsc.VectorSubcoreMesh(core_axis_name="c", subcore_axis_name="s")

    @functools.partial(
        pl.kernel,
        out_type=jax.ShapeDtypeStruct((NSEG, D), jnp.float32),
        mesh=mesh,
        scratch_types=(
            [pltpu.VMEM((CHUNK, D), jnp.float32)] * NBUF
            + [pltpu.VMEM((CHUNK,), jnp.int32)] * NBUF
            + [pltpu.VMEM((ZROWS, D), jnp.float32),
               pltpu.VMEM((16,), jnp.int32),
               pltpu.VMEM_SHARED((ACC_ROWS, D), jnp.float32)]
            + [pltpu.SemaphoreType.DMA] * (3 * NBUF)
        ),
    )
    def body(rows_hbm, ids_hbm, split_hbm, out_hbm, *refs):
        rows_v = refs[0:NBUF]
        idx_v = refs[NBUF:2 * NBUF]
        zbuf, split_v, acc = refs[2 * NBUF:2 * NBUF + 3]
        sems = refs[2 * NBUF + 3:]
        rsem = sems[0:NBUF]
        isem = sems[NBUF:2 * NBUF]
        ssem = sems[2 * NBUF:3 * NBUF]
        cid = lax.axis_index("c")
        sid = lax.axis_index("s")

        pltpu.sync_copy(split_hbm, split_v)
        r1 = split_v[...][0]
        ca = (r1 + CHUNK - 1) // CHUNK
        cb = r1 // CHUNK
        first = jnp.where(cid == 0, 0, cb)
        limit = jnp.where(cid == 0, ca, NCH - cb)
        nsteps = jnp.maximum(0, (limit - sid + NS - 1) // NS)

        half_lo = jnp.where(cid == 0, 0, SEG_SPLIT)
        half_len = jnp.where(cid == 0, HALF0, HALF1)
        lo_v = jnp.full((16,), half_lo, jnp.int32)
        len_v = jnp.full((16,), half_len, jnp.int32)

        def chunk_of(k):
            return first + sid + NS * k

        def start(k, b):
            off = pl.multiple_of(chunk_of(k) * CHUNK, CHUNK)
            pltpu.async_copy(ids_hbm.at[pl.ds(off, CHUNK)], idx_v[b], isem[b])
            pltpu.async_copy(rows_hbm.at[pl.ds(off, CHUNK)], rows_v[b], rsem[b])

        def wait(b):
            pltpu.make_async_copy(ids_hbm.at[pl.ds(0, CHUNK)], idx_v[b], isem[b]).wait()
            pltpu.make_async_copy(rows_hbm.at[pl.ds(0, CHUNK)], rows_v[b], rsem[b]).wait()

        def localize(b):
            for j in range(CHUNK // 16):
                v = idx_v[b][pl.ds(j * 16, 16)]
                loc = v - lo_v
                ok = (loc >= 0) & (loc < len_v)
                idx_v[b][pl.ds(j * 16, 16)] = jnp.where(ok, loc, len_v)

        def scatter(b):
            pltpu.sync_copy(rows_v[b], acc.at[idx_v[b]], add=True)

        @pl.when(nsteps > 0)
        def _():
            start(0, 0)

        zvec = jnp.zeros((16,), jnp.float32)

        def zstore(i, carry):
            r = i // 8
            c = lax.rem(i, 8) * 16
            zbuf[r, pl.ds(c, 16)] = zvec
            return carry

        lax.fori_loop(0, ZROWS * 8, zstore, 0)

        zstart = sid * 320
        zlen = jnp.clip(ACC_ROWS - zstart, 0, 320)
        nz = zlen // ZROWS

        def zcopy(i, carry):
            dst = pl.multiple_of(zstart + i * ZROWS, 8)
            pltpu.sync_copy(zbuf, acc.at[pl.ds(dst, ZROWS)])
            return carry

        lax.fori_loop(0, nz, zcopy, 0)
        zdst = pl.multiple_of(zstart + nz * ZROWS, 8)
        pltpu.sync_copy(zbuf.at[pl.ds(0, 40)], acc.at[pl.ds(zdst, 40)])

        plsc.subcore_barrier()

        def pair(p, carry):
            start(2 * p + 1, 1)
            wait(0)
            localize(0)
            scatter(0)

            @pl.when(2 * p + 2 < nsteps)
            def _():
                start(2 * p + 2, 0)

            wait(1)
            localize(1)
            scatter(1)
            return carry

        lax.fori_loop(0, nsteps // 2, pair, 0)

        @pl.when(lax.rem(nsteps, 2) == 1)
        def _():
            wait(0)
            localize(0)
            scatter(0)

        plsc.subcore_barrier()

        obase = pl.multiple_of(sid * 320, 8)

        @pl.when(sid < NS - 1)
        def _():
            pltpu.sync_copy(acc.at[pl.ds(obase, 320)],
                            out_hbm.at[pl.ds(pl.multiple_of(half_lo + obase, 8), 320)])

        @pl.when((sid == NS - 1) & (cid == 0))
        def _():
            pltpu.sync_copy(acc.at[pl.ds(4800, 256)],
                            out_hbm.at[pl.ds(4800, 256)])

        @pl.when((sid == NS - 1) & (cid == 1))
        def _():
            pltpu.sync_copy(acc.at[pl.ds(4800, 144)],
                            out_hbm.at[pl.ds(SEG_SPLIT + 4800, 144)])

    return body(rows, ids, split)


def kernel(output, batch):
    ids = batch.astype(jnp.int32)
    r1 = jnp.sum((ids < SEG_SPLIT).astype(jnp.int32)).astype(jnp.int32)
    split = jnp.broadcast_to(r1, (16,))
    return _sc_segment_sum(output, ids, split).reshape(-1)


# speedup vs baseline: 1.0909x; 1.0060x over previous
import jax
import jax.numpy as jnp
from jax import lax
from jax.experimental import pallas as pl
from jax.experimental.pallas import tpu as pltpu
from jax.experimental.pallas import tpu_sc as plsc
import functools

N = 320000
D = 128
NSEG = 10000

NC = 2
NS = 16
CHUNK = 320
NCH = N // CHUNK
NBUF = 2
SEG_SPLIT = 5056
HALF0 = SEG_SPLIT
HALF1 = NSEG - SEG_SPLIT
ACC_ROWS = 5064
ZROWS = 56


def _sc_segment_sum(rows, ids, split):
    mesh = plsc.VectorSubcoreMesh(core_axis_name="c", subcore_axis_name="s")

    @functools.partial(
        pl.kernel,
        out_type=jax.ShapeDtypeStruct((NSEG, D), jnp.float32),
        mesh=mesh,
        scratch_types=(
            [pltpu.VMEM((CHUNK, D), jnp.float32)] * NBUF
            + [pltpu.VMEM((CHUNK,), jnp.int32)] * NBUF
            + [pltpu.VMEM((ZROWS, D), jnp.float32),
               pltpu.VMEM((16,), jnp.int32),
               pltpu.VMEM_SHARED((ACC_ROWS, D), jnp.float32)]
            + [pltpu.SemaphoreType.DMA] * (2 * NBUF + 1)
        ),
    )
    def body(rows_hbm, ids_hbm, split_hbm, out_hbm, *refs):
        rows_v = refs[0:NBUF]
        idx_v = refs[NBUF:2 * NBUF]
        zbuf, split_v, acc = refs[2 * NBUF:2 * NBUF + 3]
        sems = refs[2 * NBUF + 3:]
        rsem = sems[0:NBUF]
        isem = sems[NBUF:2 * NBUF]
        zsem = sems[2 * NBUF]
        cid = lax.axis_index("c")
        sid = lax.axis_index("s")

        pltpu.sync_copy(split_hbm, split_v)
        r1 = split_v[...][0]
        ca = (r1 + CHUNK - 1) // CHUNK
        cb = r1 // CHUNK
        first = jnp.where(cid == 0, 0, cb)
        limit = jnp.where(cid == 0, ca, NCH - cb)
        nsteps = jnp.maximum(0, (limit - sid + NS - 1) // NS)

        half_lo = jnp.where(cid == 0, 0, SEG_SPLIT)
        half_len = jnp.where(cid == 0, HALF0, HALF1)
        lo_v = jnp.full((16,), half_lo, jnp.int32)
        len_v = jnp.full((16,), half_len, jnp.int32)

        def chunk_of(k):
            return first + sid + NS * k

        def start(k, b):
            off = pl.multiple_of(chunk_of(k) * CHUNK, CHUNK)
            pltpu.async_copy(ids_hbm.at[pl.ds(off, CHUNK)], idx_v[b], isem[b])
            pltpu.async_copy(rows_hbm.at[pl.ds(off, CHUNK)], rows_v[b], rsem[b])

        def wait(b):
            pltpu.make_async_copy(ids_hbm.at[pl.ds(0, CHUNK)], idx_v[b], isem[b]).wait()
            pltpu.make_async_copy(rows_hbm.at[pl.ds(0, CHUNK)], rows_v[b], rsem[b]).wait()

        def localize(b):
            for j in range(CHUNK // 16):
                v = idx_v[b][pl.ds(j * 16, 16)]
                loc = v - lo_v
                ok = (loc >= 0) & (loc < len_v)
                idx_v[b][pl.ds(j * 16, 16)] = jnp.where(ok, loc, len_v)

        def scatter(b):
            pltpu.sync_copy(rows_v[b], acc.at[idx_v[b]], add=True)

        @pl.when(nsteps > 0)
        def _():
            start(0, 0)

        zvec = jnp.zeros((16,), jnp.float32)

        def zrow(r, carry):
            for c in range(8):
                zbuf[r, pl.ds(c * 16, 16)] = zvec
            return carry

        lax.fori_loop(0, ZROWS, zrow, 0)

        zstart = sid * 320
        zlen = jnp.clip(ACC_ROWS - zstart, 0, 320)
        nz = zlen // ZROWS

        def zcopy(i, carry):
            dst = pl.multiple_of(zstart + i * ZROWS, 8)
            pltpu.async_copy(zbuf, acc.at[pl.ds(dst, ZROWS)], zsem)
            return carry

        lax.fori_loop(0, nz, zcopy, 0)
        zdst = pl.multiple_of(zstart + nz * ZROWS, 8)
        pltpu.async_copy(zbuf.at[pl.ds(0, 40)], acc.at[pl.ds(zdst, 40)], zsem)

        def zdrain(i, carry):
            pltpu.make_async_copy(zbuf, acc.at[pl.ds(0, ZROWS)], zsem).wait()
            return carry

        lax.fori_loop(0, nz, zdrain, 0)
        pltpu.make_async_copy(zbuf.at[pl.ds(0, 40)],
                              acc.at[pl.ds(0, 40)], zsem).wait()

        plsc.subcore_barrier()

        def pair(p, carry):
            start(2 * p + 1, 1)
            wait(0)
            localize(0)
            scatter(0)

            @pl.when(2 * p + 2 < nsteps)
            def _():
                start(2 * p + 2, 0)

            wait(1)
            localize(1)
            scatter(1)
            return carry

        lax.fori_loop(0, nsteps // 2, pair, 0)

        @pl.when(lax.rem(nsteps, 2) == 1)
        def _():
            wait(0)
            localize(0)
            scatter(0)

        plsc.subcore_barrier()

        obase = pl.multiple_of(sid * 320, 8)

        @pl.when(sid < NS - 1)
        def _():
            pltpu.sync_copy(acc.at[pl.ds(obase, 320)],
                            out_hbm.at[pl.ds(pl.multiple_of(half_lo + obase, 8), 320)])

        @pl.when((sid == NS - 1) & (cid == 0))
        def _():
            pltpu.sync_copy(acc.at[pl.ds(4800, 256)],
                            out_hbm.at[pl.ds(4800, 256)])

        @pl.when((sid == NS - 1) & (cid == 1))
        def _():
            pltpu.sync_copy(acc.at[pl.ds(4800, 144)],
                            out_hbm.at[pl.ds(SEG_SPLIT + 4800, 144)])

    return body(rows, ids, split)


def kernel(output, batch):
    ids = batch.astype(jnp.int32)
    r1 = jnp.sum((ids < SEG_SPLIT).astype(jnp.int32)).astype(jnp.int32)
    split = jnp.broadcast_to(r1, (16,))
    return _sc_segment_sum(output, ids, split).reshape(-1)
